# Initial kernel scaffold; baseline (speedup 1.0000x reference)
import jax
import jax.numpy as jnp
from jax.experimental import pallas as pl


def kernel(x, edge_index_0, edge_index_1, xnode, W0n, W0s, b0, W1n, W1s, b1, Wl0, bl0, Wa, ba, ga, be, A0, a0b, g0, bt0, A1, a1b, g1, bt1, A2, a2b, g2, bt2, WL2, bL2, g3, bt3, WL, bL):
    raise NotImplementedError("write your pallas kernel here")


# speedup vs baseline: 4.3573x; 4.3573x over previous
import functools
import jax
import jax.numpy as jnp
from jax import lax
from jax.experimental import pallas as pl
from jax.experimental.pallas import tpu as pltpu
from jax.experimental.pallas import tpu_sc as plsc

_N0, _N1, _N2 = 100000, 50000, 10000
_E0, _E1 = 500000, 160000
_D, _H, _G = 128, 128, 1
_DIN = _D + _G + 1
_W = 144
_WF = 64
_WS = 16
_ROWS = 10016
_B = 128
_NB0 = 123
_C0 = _NB0 * _B
_NB1 = 40
_C1 = _NB1 * _B
_STRIPE = _ROWS // 16
_CH = _STRIPE // 2


def _sc_agg_body(nbatch, width, clamp, with_cnt,
                 table, src_h, dst_h, zero_h, zeroc_h, ones_h, *rest):
    if with_cnt:
        (out_h, outc_h, srcv, dstv, rows, ones_v, zbuf, zbufc, acc, accc,
         sem) = rest
    else:
        out_h, srcv, dstv, rows, zbuf, acc, sem = rest
    c = lax.axis_index("c")
    s = lax.axis_index("s")
    w = c * 16 + s

    pltpu.sync_copy(src_h.at[w], srcv)
    pltpu.sync_copy(dst_h.at[w], dstv)
    pltpu.sync_copy(zero_h, zbuf)
    if with_cnt:
        pltpu.sync_copy(zeroc_h, zbufc)
        pltpu.sync_copy(ones_h, ones_v)

    def zloop(i, carry):
        pltpu.sync_copy(zbuf, acc.at[pl.ds(s * _STRIPE + i * _CH, _CH)])
        if with_cnt:
            pltpu.sync_copy(zbufc, accc.at[pl.ds(s * _STRIPE + i * _CH, _CH)])
        return carry
    lax.fori_loop(0, 2, zloop, 0)

    if clamp:
        def clamp_loop(j, carry):
            row = dstv.at[j]
            for k in range(_B // 16):
                v = row[pl.ds(k * 16, 16)]
                row[pl.ds(k * 16, 16)] = jnp.minimum(v, _N2)
            return carry
        lax.fori_loop(0, nbatch, clamp_loop, 0)

    plsc.subcore_barrier()

    def body(j, carry):
        pltpu.async_copy(table.at[srcv.at[j]], rows, sem).wait()
        pltpu.sync_copy(rows, acc.at[dstv.at[j]], add=True)
        if with_cnt:
            pltpu.sync_copy(ones_v, accc.at[dstv.at[j]], add=True)
        return carry
    lax.fori_loop(0, nbatch, body, 0)

    plsc.subcore_barrier()

    def dump(i, carry):
        r = s * _STRIPE + i * _CH
        pltpu.sync_copy(acc.at[pl.ds(r, _CH)], zbuf)
        pltpu.sync_copy(zbuf, out_h.at[c, pl.ds(r, _CH)])
        if with_cnt:
            pltpu.sync_copy(accc.at[pl.ds(r, _CH)], zbufc)
            pltpu.sync_copy(zbufc, outc_h.at[c, pl.ds(r, _CH)])
        return carry
    lax.fori_loop(0, 2, dump, 0)


def _make_sc_agg(nbatch, width, clamp, with_cnt):
    mesh = plsc.VectorSubcoreMesh(core_axis_name="c", subcore_axis_name="s")
    f32 = jnp.float32
    out_type = [jax.ShapeDtypeStruct((2, _ROWS, width), f32)]
    scratch = [
        pltpu.VMEM((nbatch, _B), jnp.int32),
        pltpu.VMEM((nbatch, _B), jnp.int32),
        pltpu.VMEM((_B, width), f32),
    ]
    if with_cnt:
        out_type.append(jax.ShapeDtypeStruct((2, _ROWS, _WS), f32))
        scratch.append(pltpu.VMEM((_B, _WS), f32))
    scratch.append(pltpu.VMEM((_CH, width), f32))
    if with_cnt:
        scratch.append(pltpu.VMEM((_CH, _WS), f32))
    scratch.append(pltpu.VMEM_SHARED((_ROWS, width), f32))
    if with_cnt:
        scratch.append(pltpu.VMEM_SHARED((_ROWS, _WS), f32))
    scratch.append(pltpu.SemaphoreType.DMA)
    return pl.kernel(
        functools.partial(_sc_agg_body, nbatch, width, clamp, with_cnt),
        out_type=out_type,
        mesh=mesh,
        scratch_types=scratch,
        compiler_params=pltpu.CompilerParams(use_tc_tiling_on_sc=False),
    )


def _tc1_body(acca_ref, accb_ref, cnt_ref, x_ref, w0n_ref, w0s_ref, b0_ref,
              w1n_ref, h1_ref, saug_ref):
    sa = acca_ref[0] + acca_ref[1]
    sb = accb_ref[0] + accb_ref[1]
    cn = cnt_ref[0] + cnt_ref[1]
    ssum = jnp.concatenate([sa, sb], axis=1)
    cnt = cn[:, 0:1]
    agg = ssum / jnp.maximum(cnt, 1.0)
    h1 = agg @ w0n_ref[...] + x_ref[...] @ w0s_ref[...] + b0_ref[...]
    h1 = jnp.maximum(h1, 0.0)
    h1_ref[...] = h1
    sv = h1 @ w1n_ref[...]
    saug_ref[...] = jnp.concatenate(
        [sv, jnp.ones_like(sv), jnp.zeros((sv.shape[0], _WS - 2), jnp.float32)],
        axis=1)


_N2F = float(_N2)


def _accum_stats(xv, stats_ref):
    part = jnp.concatenate(
        [jnp.sum(xv, axis=0, keepdims=True),
         jnp.sum(xv * xv, axis=0, keepdims=True)], axis=0)

    @pl.when(pl.program_id(0) == 0)
    def _():
        stats_ref[...] = jnp.zeros_like(stats_ref)
    stats_ref[...] += part


def _bn_from_stats(xv, stats, gamma, beta):
    m = stats[0:1] / _N2F
    v = jnp.maximum(stats[1:2] / _N2F - m * m, 0.0)
    return (xv - m) / jnp.sqrt(v + 1e-5) * gamma + beta


def _hk1_body(xnode_ref, h1_ref, acc1_ref, w1s_ref, b1_ref, wl0_ref, bl0_ref,
              wa_ref, ba_ref, xin_ref, stats_ref):
    a1sum = acc1_ref[0] + acc1_ref[1]
    agg1 = a1sum[:, 0:1] / jnp.maximum(a1sum[:, 1:2], 1.0)
    h1 = h1_ref[...]
    h2 = agg1 + h1 @ w1s_ref[...] + b1_ref[...]
    xl0 = h1 @ wl0_ref[...] + bl0_ref[...]
    xin = jnp.concatenate(
        [xnode_ref[...], h2, xl0,
         jnp.zeros((h2.shape[0], _W - _DIN), jnp.float32)], axis=1)
    z = xin @ wa_ref[...] + ba_ref[...]
    z = z - jnp.max(z, axis=1, keepdims=True)
    p = jnp.exp(z)
    p = p / jnp.sum(p, axis=1, keepdims=True)
    xin = xin * p + xin
    xin_ref[...] = xin
    _accum_stats(xin, stats_ref)


def _hk2_body(xin_ref, st_ref, ga_ref, be_ref, a0_ref, a0b_ref,
              xinb_ref, r1_ref, stats_ref):
    xinb = _bn_from_stats(xin_ref[...], st_ref[...], ga_ref[...], be_ref[...])
    xinb_ref[...] = xinb
    r1 = jnp.maximum(xinb @ a0_ref[...] + a0b_ref[...], 0.0)
    r1_ref[...] = r1
    _accum_stats(r1, stats_ref)


def _hk3_body(r1_ref, st_ref, g0_ref, bt0_ref, a1_ref, a1b_ref,
              res1_ref, r2_ref, stats_ref):
    res1 = _bn_from_stats(r1_ref[...], st_ref[...], g0_ref[...], bt0_ref[...])
    res1_ref[...] = res1
    r2 = jnp.maximum(res1 @ a1_ref[...] + a1b_ref[...], 0.0)
    r2_ref[...] = r2
    _accum_stats(r2, stats_ref)


def _hk4_body(r2_ref, st_ref, g1_ref, bt1_ref, a2_ref, a2b_ref,
              r3_ref, stats_ref):
    h = _bn_from_stats(r2_ref[...], st_ref[...], g1_ref[...], bt1_ref[...])
    r3 = jnp.maximum(h @ a2_ref[...] + a2b_ref[...], 0.0)
    r3_ref[...] = r3
    _accum_stats(r3, stats_ref)


def _hk5_body(r3_ref, st_ref, g2_ref, bt2_ref, res1_ref, wl2_ref, bl2_ref,
              r4_ref, stats_ref):
    h = _bn_from_stats(r3_ref[...], st_ref[...], g2_ref[...], bt2_ref[...])
    h = h + res1_ref[...]
    r4 = jnp.maximum(h @ wl2_ref[...] + bl2_ref[...], 0.0)
    r4_ref[...] = r4
    _accum_stats(r4, stats_ref)


def _hk6_body(r4_ref, st_ref, g3_ref, bt3_ref, xinb_ref, wl_ref, bl_ref,
              out_ref):
    h = _bn_from_stats(r4_ref[...], st_ref[...], g3_ref[...], bt3_ref[...])
    h = h + xinb_ref[...]
    out_ref[...] = h @ wl_ref[...] + bl_ref[...]


def _pad_edges(src, dst, per_tile, pad_dst):
    total = 32 * per_tile
    e = src.shape[0]
    src_p = jnp.pad(src, (0, total - e)).reshape(32, per_tile // _B, _B)
    dst_p = jnp.pad(dst, (0, total - e),
                    constant_values=pad_dst).reshape(32, per_tile // _B, _B)
    return src_p, dst_p


@jax.jit
def kernel(x, edge_index_0, edge_index_1, xnode, W0n, W0s, b0, W1n, W1s, b1,
           Wl0, bl0, Wa, ba, ga, be, A0, a0b, g0, bt0, A1, a1b, g1, bt1,
           A2, a2b, g2, bt2, WL2, bL2, g3, bt3, WL, bL):
    f32 = jnp.float32

    xa = x[:_N1, :_WF]
    xb = x[:_N1, _WF:_D]
    src0, dst0 = _pad_edges(edge_index_0[0], edge_index_0[1], _C0, _N1)
    src1, dst1 = _pad_edges(edge_index_1[0], edge_index_1[1], _C1, _N2)
    zero64 = jnp.zeros((_CH, _WF), f32)
    zero16 = jnp.zeros((_CH, _WS), f32)
    ones16 = jnp.ones((_B, _WS), f32)

    wa_p = jnp.zeros((_W, _W), f32).at[:_DIN, :_DIN].set(Wa)
    ba_p = jnp.full((1, _W), -1e30, f32).at[0, :_DIN].set(ba)
    ga_p = jnp.ones((1, _W), f32).at[0, :_DIN].set(ga)
    be_p = jnp.zeros((1, _W), f32).at[0, :_DIN].set(be)
    a0_p = jnp.zeros((_W, 256), f32).at[:_DIN].set(A0)
    wl2_p = jnp.zeros((256, _W), f32).at[:, :_DIN].set(WL2)
    bl2_p = jnp.zeros((1, _W), f32).at[0, :_DIN].set(bL2)
    g3_p = jnp.ones((1, _W), f32).at[0, :_DIN].set(g3)
    bt3_p = jnp.zeros((1, _W), f32).at[0, :_DIN].set(bt3)
    wl_p = jnp.zeros((_W, 1), f32).at[:_DIN].set(WL)

    acc_a, cnt0 = _make_sc_agg(_NB0, _WF, True, True)(
        xa, src0, dst0, zero64, zero16, ones16)
    acc_b, = _make_sc_agg(_NB0, _WF, True, False)(
        xb, src0, dst0, zero64, zero16, ones16)

    rb = 1000
    h1, saug = pl.pallas_call(
        _tc1_body,
        grid=(_N2 // rb,),
        in_specs=[
            pl.BlockSpec((2, rb, _WF), lambda i: (0, i, 0)),
            pl.BlockSpec((2, rb, _WF), lambda i: (0, i, 0)),
            pl.BlockSpec((2, rb, _WS), lambda i: (0, i, 0)),
            pl.BlockSpec((rb, _D), lambda i: (i, 0)),
            pl.BlockSpec((_D, _H), lambda i: (0, 0)),
            pl.BlockSpec((_D, _H), lambda i: (0, 0)),
            pl.BlockSpec((1, _H), lambda i: (0, 0)),
            pl.BlockSpec((_H, 1), lambda i: (0, 0)),
        ],
        out_specs=[
            pl.BlockSpec((rb, _H), lambda i: (i, 0)),
            pl.BlockSpec((rb, _WS), lambda i: (i, 0)),
        ],
        out_shape=[
            jax.ShapeDtypeStruct((_N2, _H), f32),
            jax.ShapeDtypeStruct((_N2, _WS), f32),
        ],
    )(acc_a, acc_b, cnt0, x[:_N2], W0n, W0s, b0.reshape(1, _H), W1n)

    acc1, = _make_sc_agg(_NB1, _WS, False, False)(
        saug, src1, dst1, zero16, zero16, ones16)

    grid = (_N2 // rb,)

    def row(c):
        return pl.BlockSpec((rb, c), lambda i: (i, 0))

    def full(*shape):
        return pl.BlockSpec(shape, lambda i: tuple(0 for _ in shape))

    def stats_spec(c):
        return pl.BlockSpec((2, c), lambda i: (0, 0))

    xin, st1 = pl.pallas_call(
        _hk1_body, grid=grid,
        in_specs=[row(_D), row(_H),
                  pl.BlockSpec((2, rb, _WS), lambda i: (0, i, 0)),
                  full(_H, 1), full(1, 1),
                  full(_H, 1), full(1, 1), full(_W, _W), full(1, _W)],
        out_specs=[row(_W), stats_spec(_W)],
        out_shape=[jax.ShapeDtypeStruct((_N2, _W), f32),
                   jax.ShapeDtypeStruct((2, _W), f32)],
    )(xnode, h1, acc1, W1s, b1.reshape(1, 1), Wl0, bl0.reshape(1, 1),
      wa_p, ba_p)

    xinb, r1, st2 = pl.pallas_call(
        _hk2_body, grid=grid,
        in_specs=[row(_W), stats_spec(_W), full(1, _W), full(1, _W),
                  full(_W, 256), full(1, 256)],
        out_specs=[row(_W), row(256), stats_spec(256)],
        out_shape=[jax.ShapeDtypeStruct((_N2, _W), f32),
                   jax.ShapeDtypeStruct((_N2, 256), f32),
                   jax.ShapeDtypeStruct((2, 256), f32)],
    )(xin, st1, ga_p, be_p, a0_p, a0b.reshape(1, 256))

    res1, r2, st3 = pl.pallas_call(
        _hk3_body, grid=grid,
        in_specs=[row(256), stats_spec(256), full(1, 256), full(1, 256),
                  full(256, _H), full(1, _H)],
        out_specs=[row(256), row(_H), stats_spec(_H)],
        out_shape=[jax.ShapeDtypeStruct((_N2, 256), f32),
                   jax.ShapeDtypeStruct((_N2, _H), f32),
                   jax.ShapeDtypeStruct((2, _H), f32)],
    )(r1, st2, g0.reshape(1, 256), bt0.reshape(1, 256), A1,
      a1b.reshape(1, _H))

    r3, st4 = pl.pallas_call(
        _hk4_body, grid=grid,
        in_specs=[row(_H), stats_spec(_H), full(1, _H), full(1, _H),
                  full(_H, 256), full(1, 256)],
        out_specs=[row(256), stats_spec(256)],
        out_shape=[jax.ShapeDtypeStruct((_N2, 256), f32),
                   jax.ShapeDtypeStruct((2, 256), f32)],
    )(r2, st3, g1.reshape(1, _H), bt1.reshape(1, _H), A2,
      a2b.reshape(1, 256))

    r4, st5 = pl.pallas_call(
        _hk5_body, grid=grid,
        in_specs=[row(256), stats_spec(256), full(1, 256), full(1, 256),
                  row(256), full(256, _W), full(1, _W)],
        out_specs=[row(_W), stats_spec(_W)],
        out_shape=[jax.ShapeDtypeStruct((_N2, _W), f32),
                   jax.ShapeDtypeStruct((2, _W), f32)],
    )(r3, st4, g2.reshape(1, 256), bt2.reshape(1, 256), res1, wl2_p, bl2_p)

    out = pl.pallas_call(
        _hk6_body, grid=grid,
        in_specs=[row(_W), stats_spec(_W), full(1, _W), full(1, _W),
                  row(_W), full(_W, 1), full(1, 1)],
        out_specs=pl.BlockSpec((rb, 1), lambda i: (i, 0)),
        out_shape=jax.ShapeDtypeStruct((_N2, 1), f32),
    )(r4, st5, g3_p, bt3_p, xinb, wl_p, bL.reshape(1, 1))
    return out


# speedup vs baseline: 8.6186x; 1.9780x over previous
import functools
import jax
import jax.numpy as jnp
from jax import lax
from jax.experimental import pallas as pl
from jax.experimental.pallas import tpu as pltpu
from jax.experimental.pallas import tpu_sc as plsc

_N0, _N1, _N2 = 100000, 50000, 10000
_E0, _E1 = 500000, 160000
_D, _H, _G = 128, 128, 1
_DIN = _D + _G + 1
_W = 144
_WF = 64
_WS = 16
_ROWS = 10016
_B = 128
_NB0 = 123
_C0 = _NB0 * _B
_NB1 = 40
_C1 = _NB1 * _B
_STRIPE = _ROWS // 16
_CH = _STRIPE // 2


def _sc_agg_body(nbatch, width, compact, mode,
                 table, src_h, dst_h, zero_h, ones_h,
                 out_h, *rest):
    rest = list(rest)
    rsrc = srcv = rest.pop(0)
    dstv = rest.pop(0)
    csrc = rest.pop(0) if (compact and mode == "feat") else None
    rows = rest.pop(0)
    zbuf = rest.pop(0)
    acc = rest.pop(0)
    sem = rest.pop(0)
    c = lax.axis_index("c")
    s = lax.axis_index("s")
    w = c * 16 + s

    if mode == "feat":
        pltpu.sync_copy(src_h.at[w], srcv)
    else:
        pltpu.sync_copy(ones_h, rows)
    if compact:
        pltpu.sync_copy(dst_h.at[w], dstv.at[pl.ds(0, nbatch)])
    else:
        pltpu.sync_copy(dst_h.at[w], dstv)
    pltpu.sync_copy(zero_h, zbuf)

    def zloop(i, carry):
        pltpu.sync_copy(zbuf, acc.at[pl.ds(s * _STRIPE + i * _CH, _CH)])
        return carry
    lax.fori_loop(0, 2, zloop, 0)

    if compact:
        def cloop(j, cntv):
            drow = dstv.at[j]
            if mode == "feat":
                srow = rsrc.at[j]
            for k in range(_B // 16):
                d16 = drow[pl.ds(k * 16, 16)]
                m = d16 < _N2
                incl = plsc.cumsum(m.astype(jnp.int32))
                pos = jnp.where(m, cntv + incl - 1, nbatch * _B + _B - 1)
                if mode == "feat":
                    s16 = srow[pl.ds(k * 16, 16)]
                    plsc.store_scatter(csrc, [pos], s16)
                plsc.store_scatter(dstv, [pos >> 7, pos & (_B - 1)], d16)
                cntv = cntv + plsc.all_reduce_population_count(m)
            return cntv
        cntv = lax.fori_loop(0, nbatch, cloop, jnp.zeros((16,), jnp.int32))
        cnt = cntv[0]

        def ploop(i, cend):
            del i
            if mode == "feat":
                csrc[pl.ds(cend, 16)] = jnp.zeros((16,), jnp.int32)
            pp = cend + lax.iota(jnp.int32, 16)
            plsc.store_scatter(dstv, [pp >> 7, pp & (_B - 1)],
                               jnp.full((16,), _N2, jnp.int32))
            return cend + 16
        lax.fori_loop(0, _B // 16, ploop, cnt)

        nb = (cnt + (_B - 1)) // _B
    else:
        nb = nbatch

    plsc.subcore_barrier()

    def body(j, carry):
        if mode == "feat":
            if compact:
                idx = csrc.at[pl.ds(j * _B, _B)]
            else:
                idx = srcv.at[j]
            pltpu.async_copy(table.at[idx], rows, sem).wait()
        pltpu.sync_copy(rows, acc.at[dstv.at[j]], add=True)
        return carry
    lax.fori_loop(0, nb, body, 0)

    plsc.subcore_barrier()

    def dump(i, carry):
        r = s * _STRIPE + i * _CH
        pltpu.sync_copy(acc.at[pl.ds(r, _CH)], zbuf)
        pltpu.sync_copy(zbuf, out_h.at[c, pl.ds(r, _CH)])
        return carry
    lax.fori_loop(0, 2, dump, 0)


def _make_sc_agg(nbatch, width, compact, mode):
    mesh = plsc.VectorSubcoreMesh(core_axis_name="c", subcore_axis_name="s")
    f32 = jnp.float32
    ne = nbatch * _B
    out_type = [jax.ShapeDtypeStruct((2, _ROWS, width), f32)]
    scratch = [
        pltpu.VMEM((nbatch, _B), jnp.int32),
        pltpu.VMEM((nbatch + (1 if compact else 0), _B), jnp.int32),
    ]
    if compact and mode == "feat":
        scratch.append(pltpu.VMEM((ne + _B,), jnp.int32))
    scratch.append(pltpu.VMEM((_B, width), f32))
    scratch.append(pltpu.VMEM((_CH, width), f32))
    scratch.append(pltpu.VMEM_SHARED((_ROWS, width), f32))
    scratch.append(pltpu.SemaphoreType.DMA)
    return pl.kernel(
        functools.partial(_sc_agg_body, nbatch, width, compact, mode),
        out_type=out_type,
        mesh=mesh,
        scratch_types=scratch,
        compiler_params=pltpu.CompilerParams(
            use_tc_tiling_on_sc=False,
            needs_layout_passes=not compact,
        ),
    )


def _tc1_body(acca_ref, accb_ref, cnt_ref, x_ref, w0n_ref, w0s_ref, b0_ref,
              w1n_ref, h1_ref, saug_ref):
    sa = acca_ref[0] + acca_ref[1]
    sb = accb_ref[0] + accb_ref[1]
    cn = cnt_ref[0] + cnt_ref[1]
    ssum = jnp.concatenate([sa, sb], axis=1)
    cnt = cn[:, 0:1]
    agg = ssum / jnp.maximum(cnt, 1.0)
    h1 = agg @ w0n_ref[...] + x_ref[...] @ w0s_ref[...] + b0_ref[...]
    h1 = jnp.maximum(h1, 0.0)
    h1_ref[...] = h1
    sv = h1 @ w1n_ref[...]
    saug_ref[...] = jnp.concatenate(
        [sv, jnp.ones_like(sv), jnp.zeros((sv.shape[0], _WS - 2), jnp.float32)],
        axis=1)


_N2F = float(_N2)


def _accum_stats(xv, stats_ref):
    part = jnp.concatenate(
        [jnp.sum(xv, axis=0, keepdims=True),
         jnp.sum(xv * xv, axis=0, keepdims=True)], axis=0)

    @pl.when(pl.program_id(0) == 0)
    def _():
        stats_ref[...] = jnp.zeros_like(stats_ref)
    stats_ref[...] += part


def _bn_from_stats(xv, stats, gamma, beta):
    m = stats[0:1] / _N2F
    v = jnp.maximum(stats[1:2] / _N2F - m * m, 0.0)
    return (xv - m) / jnp.sqrt(v + 1e-5) * gamma + beta


def _hk1_body(xnode_ref, h1_ref, acc1_ref, w1s_ref, b1_ref, wl0_ref, bl0_ref,
              wa_ref, ba_ref, xin_ref, stats_ref):
    a1sum = acc1_ref[0] + acc1_ref[1]
    agg1 = a1sum[:, 0:1] / jnp.maximum(a1sum[:, 1:2], 1.0)
    h1 = h1_ref[...]
    h2 = agg1 + h1 @ w1s_ref[...] + b1_ref[...]
    xl0 = h1 @ wl0_ref[...] + bl0_ref[...]
    xin = jnp.concatenate(
        [xnode_ref[...], h2, xl0,
         jnp.zeros((h2.shape[0], _W - _DIN), jnp.float32)], axis=1)
    z = xin @ wa_ref[...] + ba_ref[...]
    z = z - jnp.max(z, axis=1, keepdims=True)
    p = jnp.exp(z)
    p = p / jnp.sum(p, axis=1, keepdims=True)
    xin = xin * p + xin
    xin_ref[...] = xin
    _accum_stats(xin, stats_ref)


def _hk2_body(xin_ref, st_ref, ga_ref, be_ref, a0_ref, a0b_ref,
              xinb_ref, r1_ref, stats_ref):
    xinb = _bn_from_stats(xin_ref[...], st_ref[...], ga_ref[...], be_ref[...])
    xinb_ref[...] = xinb
    r1 = jnp.maximum(xinb @ a0_ref[...] + a0b_ref[...], 0.0)
    r1_ref[...] = r1
    _accum_stats(r1, stats_ref)


def _hk3_body(r1_ref, st_ref, g0_ref, bt0_ref, a1_ref, a1b_ref,
              res1_ref, r2_ref, stats_ref):
    res1 = _bn_from_stats(r1_ref[...], st_ref[...], g0_ref[...], bt0_ref[...])
    res1_ref[...] = res1
    r2 = jnp.maximum(res1 @ a1_ref[...] + a1b_ref[...], 0.0)
    r2_ref[...] = r2
    _accum_stats(r2, stats_ref)


def _hk4_body(r2_ref, st_ref, g1_ref, bt1_ref, a2_ref, a2b_ref,
              r3_ref, stats_ref):
    h = _bn_from_stats(r2_ref[...], st_ref[...], g1_ref[...], bt1_ref[...])
    r3 = jnp.maximum(h @ a2_ref[...] + a2b_ref[...], 0.0)
    r3_ref[...] = r3
    _accum_stats(r3, stats_ref)


def _hk5_body(r3_ref, st_ref, g2_ref, bt2_ref, res1_ref, wl2_ref, bl2_ref,
              r4_ref, stats_ref):
    h = _bn_from_stats(r3_ref[...], st_ref[...], g2_ref[...], bt2_ref[...])
    h = h + res1_ref[...]
    r4 = jnp.maximum(h @ wl2_ref[...] + bl2_ref[...], 0.0)
    r4_ref[...] = r4
    _accum_stats(r4, stats_ref)


def _hk6_body(r4_ref, st_ref, g3_ref, bt3_ref, xinb_ref, wl_ref, bl_ref,
              out_ref):
    h = _bn_from_stats(r4_ref[...], st_ref[...], g3_ref[...], bt3_ref[...])
    h = h + xinb_ref[...]
    out_ref[...] = h @ wl_ref[...] + bl_ref[...]


def _pad_edges(src, dst, per_tile, pad_dst):
    total = 32 * per_tile
    e = src.shape[0]
    src_p = jnp.pad(src, (0, total - e)).reshape(32, per_tile // _B, _B)
    dst_p = jnp.pad(dst, (0, total - e),
                    constant_values=pad_dst).reshape(32, per_tile // _B, _B)
    return src_p, dst_p


@jax.jit
def kernel(x, edge_index_0, edge_index_1, xnode, W0n, W0s, b0, W1n, W1s, b1,
           Wl0, bl0, Wa, ba, ga, be, A0, a0b, g0, bt0, A1, a1b, g1, bt1,
           A2, a2b, g2, bt2, WL2, bL2, g3, bt3, WL, bL):
    f32 = jnp.float32

    xa = x[:_N1, :_WF]
    xb = x[:_N1, _WF:_D]
    src0, dst0 = _pad_edges(edge_index_0[0], edge_index_0[1], _C0, _N1)
    src1, dst1 = _pad_edges(edge_index_1[0], edge_index_1[1], _C1, _N2)
    zero64 = jnp.zeros((_CH, _WF), f32)
    zero16 = jnp.zeros((_CH, _WS), f32)
    ones16 = jnp.ones((_B, _WS), f32)

    wa_p = jnp.zeros((_W, _W), f32).at[:_DIN, :_DIN].set(Wa)
    ba_p = jnp.full((1, _W), -1e30, f32).at[0, :_DIN].set(ba)
    ga_p = jnp.ones((1, _W), f32).at[0, :_DIN].set(ga)
    be_p = jnp.zeros((1, _W), f32).at[0, :_DIN].set(be)
    a0_p = jnp.zeros((_W, 256), f32).at[:_DIN].set(A0)
    wl2_p = jnp.zeros((256, _W), f32).at[:, :_DIN].set(WL2)
    bl2_p = jnp.zeros((1, _W), f32).at[0, :_DIN].set(bL2)
    g3_p = jnp.ones((1, _W), f32).at[0, :_DIN].set(g3)
    bt3_p = jnp.zeros((1, _W), f32).at[0, :_DIN].set(bt3)
    wl_p = jnp.zeros((_W, 1), f32).at[:_DIN].set(WL)

    acc_a, = _make_sc_agg(_NB0, _WF, True, "feat")(
        xa, src0, dst0, zero64, ones16)
    acc_b, = _make_sc_agg(_NB0, _WF, True, "feat")(
        xb, src0, dst0, zero64, ones16)
    cnt0, = _make_sc_agg(_NB0, _WS, True, "cnt")(
        ones16, src0, dst0, zero16, ones16)

    rb = 1000
    h1, saug = pl.pallas_call(
        _tc1_body,
        grid=(_N2 // rb,),
        in_specs=[
            pl.BlockSpec((2, rb, _WF), lambda i: (0, i, 0)),
            pl.BlockSpec((2, rb, _WF), lambda i: (0, i, 0)),
            pl.BlockSpec((2, rb, _WS), lambda i: (0, i, 0)),
            pl.BlockSpec((rb, _D), lambda i: (i, 0)),
            pl.BlockSpec((_D, _H), lambda i: (0, 0)),
            pl.BlockSpec((_D, _H), lambda i: (0, 0)),
            pl.BlockSpec((1, _H), lambda i: (0, 0)),
            pl.BlockSpec((_H, 1), lambda i: (0, 0)),
        ],
        out_specs=[
            pl.BlockSpec((rb, _H), lambda i: (i, 0)),
            pl.BlockSpec((rb, _WS), lambda i: (i, 0)),
        ],
        out_shape=[
            jax.ShapeDtypeStruct((_N2, _H), f32),
            jax.ShapeDtypeStruct((_N2, _WS), f32),
        ],
    )(acc_a, acc_b, cnt0, x[:_N2], W0n, W0s, b0.reshape(1, _H), W1n)

    acc1, = _make_sc_agg(_NB1, _WS, False, "feat")(
        saug, src1, dst1, zero16, ones16)

    grid = (_N2 // rb,)

    def row(c):
        return pl.BlockSpec((rb, c), lambda i: (i, 0))

    def full(*shape):
        return pl.BlockSpec(shape, lambda i: tuple(0 for _ in shape))

    def stats_spec(c):
        return pl.BlockSpec((2, c), lambda i: (0, 0))

    xin, st1 = pl.pallas_call(
        _hk1_body, grid=grid,
        in_specs=[row(_D), row(_H),
                  pl.BlockSpec((2, rb, _WS), lambda i: (0, i, 0)),
                  full(_H, 1), full(1, 1),
                  full(_H, 1), full(1, 1), full(_W, _W), full(1, _W)],
        out_specs=[row(_W), stats_spec(_W)],
        out_shape=[jax.ShapeDtypeStruct((_N2, _W), f32),
                   jax.ShapeDtypeStruct((2, _W), f32)],
    )(xnode, h1, acc1, W1s, b1.reshape(1, 1), Wl0, bl0.reshape(1, 1),
      wa_p, ba_p)

    xinb, r1, st2 = pl.pallas_call(
        _hk2_body, grid=grid,
        in_specs=[row(_W), stats_spec(_W), full(1, _W), full(1, _W),
                  full(_W, 256), full(1, 256)],
        out_specs=[row(_W), row(256), stats_spec(256)],
        out_shape=[jax.ShapeDtypeStruct((_N2, _W), f32),
                   jax.ShapeDtypeStruct((_N2, 256), f32),
                   jax.ShapeDtypeStruct((2, 256), f32)],
    )(xin, st1, ga_p, be_p, a0_p, a0b.reshape(1, 256))

    res1, r2, st3 = pl.pallas_call(
        _hk3_body, grid=grid,
        in_specs=[row(256), stats_spec(256), full(1, 256), full(1, 256),
                  full(256, _H), full(1, _H)],
        out_specs=[row(256), row(_H), stats_spec(_H)],
        out_shape=[jax.ShapeDtypeStruct((_N2, 256), f32),
                   jax.ShapeDtypeStruct((_N2, _H), f32),
                   jax.ShapeDtypeStruct((2, _H), f32)],
    )(r1, st2, g0.reshape(1, 256), bt0.reshape(1, 256), A1,
      a1b.reshape(1, _H))

    r3, st4 = pl.pallas_call(
        _hk4_body, grid=grid,
        in_specs=[row(_H), stats_spec(_H), full(1, _H), full(1, _H),
                  full(_H, 256), full(1, 256)],
        out_specs=[row(256), stats_spec(256)],
        out_shape=[jax.ShapeDtypeStruct((_N2, 256), f32),
                   jax.ShapeDtypeStruct((2, 256), f32)],
    )(r2, st3, g1.reshape(1, _H), bt1.reshape(1, _H), A2,
      a2b.reshape(1, 256))

    r4, st5 = pl.pallas_call(
        _hk5_body, grid=grid,
        in_specs=[row(256), stats_spec(256), full(1, 256), full(1, 256),
                  row(256), full(256, _W), full(1, _W)],
        out_specs=[row(_W), stats_spec(_W)],
        out_shape=[jax.ShapeDtypeStruct((_N2, _W), f32),
                   jax.ShapeDtypeStruct((2, _W), f32)],
    )(r3, st4, g2.reshape(1, 256), bt2.reshape(1, 256), res1, wl2_p, bl2_p)

    out = pl.pallas_call(
        _hk6_body, grid=grid,
        in_specs=[row(_W), stats_spec(_W), full(1, _W), full(1, _W),
                  row(_W), full(_W, 1), full(1, 1)],
        out_specs=pl.BlockSpec((rb, 1), lambda i: (i, 0)),
        out_shape=jax.ShapeDtypeStruct((_N2, 1), f32),
    )(r4, st5, g3_p, bt3_p, xinb, wl_p, bL.reshape(1, 1))
    return out


# speedup vs baseline: 9.4162x; 1.0925x over previous
import functools
import jax
import jax.numpy as jnp
from jax import lax
from jax.experimental import pallas as pl
from jax.experimental.pallas import tpu as pltpu
from jax.experimental.pallas import tpu_sc as plsc

_N0, _N1, _N2 = 100000, 50000, 10000
_E0, _E1 = 500000, 160000
_D, _H, _G = 128, 128, 1
_DIN = _D + _G + 1
_W = 144
_WF = 64
_WS = 16
_ROWS = 10016
_B = 128
_NB0 = 123
_C0 = _NB0 * _B
_NB1 = 40
_C1 = _NB1 * _B
_STRIPE = _ROWS // 16
_NBUF = 2
_CH = _STRIPE // 2


def _sc_agg_body(nbatch, width, compact, mode,
                 table, src_h, dst_h, zero_h, ones_h,
                 out_h, *rest):
    rest = list(rest)
    rsrc = srcv = rest.pop(0)
    dstv = rest.pop(0)
    csrc = rest.pop(0) if (compact and mode == "feat") else None
    rows = rest.pop(0)
    zbuf = rest.pop(0)
    acc = rest.pop(0)
    sem = rest.pop(0)
    c = lax.axis_index("c")
    s = lax.axis_index("s")
    w = c * 16 + s

    if mode == "feat":
        pltpu.sync_copy(src_h.at[w], srcv)
    else:
        pltpu.sync_copy(ones_h, rows)

    if compact:
        pltpu.sync_copy(dst_h.at[w], dstv.at[pl.ds(0, nbatch)])
    else:
        pltpu.sync_copy(dst_h.at[w], dstv)
    pltpu.sync_copy(zero_h, zbuf)

    def zloop(i, carry):
        pltpu.sync_copy(zbuf, acc.at[pl.ds(s * _STRIPE + i * _CH, _CH)])
        return carry
    lax.fori_loop(0, 2, zloop, 0)

    if compact:
        def cloop(j, cntv):
            drow = dstv.at[j]
            if mode == "feat":
                srow = rsrc.at[j]
            for k in range(_B // 16):
                d16 = drow[pl.ds(k * 16, 16)]
                m = d16 < _N2
                incl = plsc.cumsum(m.astype(jnp.int32))
                pos = jnp.where(m, cntv + incl - 1, nbatch * _B + _B - 1)
                if mode == "feat":
                    s16 = srow[pl.ds(k * 16, 16)]
                    plsc.store_scatter(csrc, [pos], s16)
                plsc.store_scatter(dstv, [pos >> 7, pos & (_B - 1)], d16)
                cntv = cntv + plsc.all_reduce_population_count(m)
            return cntv
        cntv = lax.fori_loop(0, nbatch, cloop, jnp.zeros((16,), jnp.int32))
        cnt = cntv[0]

        def ploop(i, cend):
            del i
            if mode == "feat":
                csrc[pl.ds(cend, 16)] = jnp.zeros((16,), jnp.int32)
            pp = cend + lax.iota(jnp.int32, 16)
            plsc.store_scatter(dstv, [pp >> 7, pp & (_B - 1)],
                               jnp.full((16,), _N2, jnp.int32))
            return cend + 16
        lax.fori_loop(0, _B // 16, ploop, cnt)

        nb = (cnt + (_B - 1)) // _B
    else:
        nb = nbatch

    plsc.subcore_barrier()

    if mode == "feat":
        def gidx(j):
            if compact:
                return csrc.at[pl.ds(j * _B, _B)]
            return srcv.at[j]

        def fire(j):
            pltpu.async_copy(table.at[gidx(j)], rows.at[j & (_NBUF - 1)], sem)

        def prol(j, carry):
            @pl.when(j < nb)
            def _():
                fire(j)
            return carry
        lax.fori_loop(0, _NBUF, prol, 0)

        def body(j, carry):
            b = j & (_NBUF - 1)
            pltpu.make_async_copy(table.at[gidx(j)], rows.at[b], sem).wait()
            pltpu.sync_copy(rows.at[b], acc.at[dstv.at[j]], add=True)

            @pl.when(j + _NBUF < nb)
            def _():
                fire(j + _NBUF)
            return carry
        lax.fori_loop(0, nb, body, 0)
    else:
        def body(j, carry):
            pltpu.sync_copy(rows, acc.at[dstv.at[j]], add=True)
            return carry
        lax.fori_loop(0, nb, body, 0)

    plsc.subcore_barrier()

    def dump(i, carry):
        r = s * _STRIPE + i * _CH
        pltpu.sync_copy(acc.at[pl.ds(r, _CH)], zbuf)
        pltpu.sync_copy(zbuf, out_h.at[c, pl.ds(r, _CH)])
        return carry
    lax.fori_loop(0, 2, dump, 0)


def _make_sc_agg(nbatch, width, compact, mode):
    mesh = plsc.VectorSubcoreMesh(core_axis_name="c", subcore_axis_name="s")
    f32 = jnp.float32
    ne = nbatch * _B
    out_type = [jax.ShapeDtypeStruct((2, _ROWS, width), f32)]
    scratch = [
        pltpu.VMEM((nbatch, _B), jnp.int32),
        pltpu.VMEM((nbatch + (1 if compact else 0), _B), jnp.int32),
    ]
    if compact and mode == "feat":
        scratch.append(pltpu.VMEM((ne + _B,), jnp.int32))
    scratch.append(pltpu.VMEM((_NBUF, _B, width) if mode == "feat"
                              else (_B, width), f32))
    scratch.append(pltpu.VMEM((_CH, width), f32))
    scratch.append(pltpu.VMEM_SHARED((_ROWS, width), f32))
    scratch.append(pltpu.SemaphoreType.DMA)
    return pl.kernel(
        functools.partial(_sc_agg_body, nbatch, width, compact, mode),
        out_type=out_type,
        mesh=mesh,
        scratch_types=scratch,
        compiler_params=pltpu.CompilerParams(
            use_tc_tiling_on_sc=False,
            needs_layout_passes=not compact,
        ),
    )


def _tc1_body(acca_ref, accb_ref, cnt_ref, x_ref, w0n_ref, w0s_ref, b0_ref,
              w1n_ref, h1_ref, saug_ref):
    sa = acca_ref[0] + acca_ref[1]
    sb = accb_ref[0] + accb_ref[1]
    cn = cnt_ref[0] + cnt_ref[1]
    ssum = jnp.concatenate([sa, sb], axis=1)
    cnt = cn[:, 0:1]
    agg = ssum / jnp.maximum(cnt, 1.0)
    h1 = agg @ w0n_ref[...] + x_ref[...] @ w0s_ref[...] + b0_ref[...]
    h1 = jnp.maximum(h1, 0.0)
    h1_ref[...] = h1
    sv = h1 @ w1n_ref[...]
    saug_ref[...] = jnp.concatenate(
        [sv, jnp.ones_like(sv), jnp.zeros((sv.shape[0], _WS - 2), jnp.float32)],
        axis=1)


_N2F = float(_N2)


def _accum_stats(xv, stats_ref):
    part = jnp.concatenate(
        [jnp.sum(xv, axis=0, keepdims=True),
         jnp.sum(xv * xv, axis=0, keepdims=True)], axis=0)

    @pl.when(pl.program_id(0) == 0)
    def _():
        stats_ref[...] = jnp.zeros_like(stats_ref)
    stats_ref[...] += part


def _bn_from_stats(xv, stats, gamma, beta):
    m = stats[0:1] / _N2F
    v = jnp.maximum(stats[1:2] / _N2F - m * m, 0.0)
    return (xv - m) / jnp.sqrt(v + 1e-5) * gamma + beta


def _hk1_body(xnode_ref, h1_ref, acc1_ref, w1s_ref, b1_ref, wl0_ref, bl0_ref,
              wa_ref, ba_ref, xin_ref, stats_ref):
    a1sum = acc1_ref[0] + acc1_ref[1]
    agg1 = a1sum[:, 0:1] / jnp.maximum(a1sum[:, 1:2], 1.0)
    h1 = h1_ref[...]
    h2 = agg1 + h1 @ w1s_ref[...] + b1_ref[...]
    xl0 = h1 @ wl0_ref[...] + bl0_ref[...]
    xin = jnp.concatenate(
        [xnode_ref[...], h2, xl0,
         jnp.zeros((h2.shape[0], _W - _DIN), jnp.float32)], axis=1)
    z = xin @ wa_ref[...] + ba_ref[...]
    z = z - jnp.max(z, axis=1, keepdims=True)
    p = jnp.exp(z)
    p = p / jnp.sum(p, axis=1, keepdims=True)
    xin = xin * p + xin
    xin_ref[...] = xin
    _accum_stats(xin, stats_ref)


def _hk2_body(xin_ref, st_ref, ga_ref, be_ref, a0_ref, a0b_ref,
              xinb_ref, r1_ref, stats_ref):
    xinb = _bn_from_stats(xin_ref[...], st_ref[...], ga_ref[...], be_ref[...])
    xinb_ref[...] = xinb
    r1 = jnp.maximum(xinb @ a0_ref[...] + a0b_ref[...], 0.0)
    r1_ref[...] = r1
    _accum_stats(r1, stats_ref)


def _hk3_body(r1_ref, st_ref, g0_ref, bt0_ref, a1_ref, a1b_ref,
              res1_ref, r2_ref, stats_ref):
    res1 = _bn_from_stats(r1_ref[...], st_ref[...], g0_ref[...], bt0_ref[...])
    res1_ref[...] = res1
    r2 = jnp.maximum(res1 @ a1_ref[...] + a1b_ref[...], 0.0)
    r2_ref[...] = r2
    _accum_stats(r2, stats_ref)


def _hk4_body(r2_ref, st_ref, g1_ref, bt1_ref, a2_ref, a2b_ref,
              r3_ref, stats_ref):
    h = _bn_from_stats(r2_ref[...], st_ref[...], g1_ref[...], bt1_ref[...])
    r3 = jnp.maximum(h @ a2_ref[...] + a2b_ref[...], 0.0)
    r3_ref[...] = r3
    _accum_stats(r3, stats_ref)


def _hk5_body(r3_ref, st_ref, g2_ref, bt2_ref, res1_ref, wl2_ref, bl2_ref,
              r4_ref, stats_ref):
    h = _bn_from_stats(r3_ref[...], st_ref[...], g2_ref[...], bt2_ref[...])
    h = h + res1_ref[...]
    r4 = jnp.maximum(h @ wl2_ref[...] + bl2_ref[...], 0.0)
    r4_ref[...] = r4
    _accum_stats(r4, stats_ref)


def _hk6_body(r4_ref, st_ref, g3_ref, bt3_ref, xinb_ref, wl_ref, bl_ref,
              out_ref):
    h = _bn_from_stats(r4_ref[...], st_ref[...], g3_ref[...], bt3_ref[...])
    h = h + xinb_ref[...]
    out_ref[...] = h @ wl_ref[...] + bl_ref[...]


def _pad_edges(src, dst, per_tile, pad_dst):
    total = 32 * per_tile
    e = src.shape[0]
    src_p = jnp.pad(src, (0, total - e)).reshape(32, per_tile // _B, _B)
    dst_p = jnp.pad(dst, (0, total - e),
                    constant_values=pad_dst).reshape(32, per_tile // _B, _B)
    return src_p, dst_p


@jax.jit
def kernel(x, edge_index_0, edge_index_1, xnode, W0n, W0s, b0, W1n, W1s, b1,
           Wl0, bl0, Wa, ba, ga, be, A0, a0b, g0, bt0, A1, a1b, g1, bt1,
           A2, a2b, g2, bt2, WL2, bL2, g3, bt3, WL, bL):
    f32 = jnp.float32

    xa = x[:_N1, :_WF]
    xb = x[:_N1, _WF:_D]
    src0, dst0 = _pad_edges(edge_index_0[0], edge_index_0[1], _C0, _N1)
    src1, dst1 = _pad_edges(edge_index_1[0], edge_index_1[1], _C1, _N2)
    zero64 = jnp.zeros((_CH, _WF), f32)
    zero16 = jnp.zeros((_CH, _WS), f32)
    ones16 = jnp.ones((_B, _WS), f32)

    wa_p = jnp.zeros((_W, _W), f32).at[:_DIN, :_DIN].set(Wa)
    ba_p = jnp.full((1, _W), -1e30, f32).at[0, :_DIN].set(ba)
    ga_p = jnp.ones((1, _W), f32).at[0, :_DIN].set(ga)
    be_p = jnp.zeros((1, _W), f32).at[0, :_DIN].set(be)
    a0_p = jnp.zeros((_W, 256), f32).at[:_DIN].set(A0)
    wl2_p = jnp.zeros((256, _W), f32).at[:, :_DIN].set(WL2)
    bl2_p = jnp.zeros((1, _W), f32).at[0, :_DIN].set(bL2)
    g3_p = jnp.ones((1, _W), f32).at[0, :_DIN].set(g3)
    bt3_p = jnp.zeros((1, _W), f32).at[0, :_DIN].set(bt3)
    wl_p = jnp.zeros((_W, 1), f32).at[:_DIN].set(WL)

    acc_a, = _make_sc_agg(_NB0, _WF, True, "feat")(
        xa, src0, dst0, zero64, ones16)
    acc_b, = _make_sc_agg(_NB0, _WF, True, "feat")(
        xb, src0, dst0, zero64, ones16)
    cnt0, = _make_sc_agg(_NB0, _WS, True, "cnt")(
        ones16, src0, dst0, zero16, ones16)

    rb = 1000
    h1, saug = pl.pallas_call(
        _tc1_body,
        grid=(_N2 // rb,),
        in_specs=[
            pl.BlockSpec((2, rb, _WF), lambda i: (0, i, 0)),
            pl.BlockSpec((2, rb, _WF), lambda i: (0, i, 0)),
            pl.BlockSpec((2, rb, _WS), lambda i: (0, i, 0)),
            pl.BlockSpec((rb, _D), lambda i: (i, 0)),
            pl.BlockSpec((_D, _H), lambda i: (0, 0)),
            pl.BlockSpec((_D, _H), lambda i: (0, 0)),
            pl.BlockSpec((1, _H), lambda i: (0, 0)),
            pl.BlockSpec((_H, 1), lambda i: (0, 0)),
        ],
        out_specs=[
            pl.BlockSpec((rb, _H), lambda i: (i, 0)),
            pl.BlockSpec((rb, _WS), lambda i: (i, 0)),
        ],
        out_shape=[
            jax.ShapeDtypeStruct((_N2, _H), f32),
            jax.ShapeDtypeStruct((_N2, _WS), f32),
        ],
    )(acc_a, acc_b, cnt0, x[:_N2], W0n, W0s, b0.reshape(1, _H), W1n)

    acc1, = _make_sc_agg(_NB1, _WS, False, "feat")(
        saug, src1, dst1, zero16, ones16)

    grid = (_N2 // rb,)

    def row(c):
        return pl.BlockSpec((rb, c), lambda i: (i, 0))

    def full(*shape):
        return pl.BlockSpec(shape, lambda i: tuple(0 for _ in shape))

    def stats_spec(c):
        return pl.BlockSpec((2, c), lambda i: (0, 0))

    xin, st1 = pl.pallas_call(
        _hk1_body, grid=grid,
        in_specs=[row(_D), row(_H),
                  pl.BlockSpec((2, rb, _WS), lambda i: (0, i, 0)),
                  full(_H, 1), full(1, 1),
                  full(_H, 1), full(1, 1), full(_W, _W), full(1, _W)],
        out_specs=[row(_W), stats_spec(_W)],
        out_shape=[jax.ShapeDtypeStruct((_N2, _W), f32),
                   jax.ShapeDtypeStruct((2, _W), f32)],
    )(xnode, h1, acc1, W1s, b1.reshape(1, 1), Wl0, bl0.reshape(1, 1),
      wa_p, ba_p)

    xinb, r1, st2 = pl.pallas_call(
        _hk2_body, grid=grid,
        in_specs=[row(_W), stats_spec(_W), full(1, _W), full(1, _W),
                  full(_W, 256), full(1, 256)],
        out_specs=[row(_W), row(256), stats_spec(256)],
        out_shape=[jax.ShapeDtypeStruct((_N2, _W), f32),
                   jax.ShapeDtypeStruct((_N2, 256), f32),
                   jax.ShapeDtypeStruct((2, 256), f32)],
    )(xin, st1, ga_p, be_p, a0_p, a0b.reshape(1, 256))

    res1, r2, st3 = pl.pallas_call(
        _hk3_body, grid=grid,
        in_specs=[row(256), stats_spec(256), full(1, 256), full(1, 256),
                  full(256, _H), full(1, _H)],
        out_specs=[row(256), row(_H), stats_spec(_H)],
        out_shape=[jax.ShapeDtypeStruct((_N2, 256), f32),
                   jax.ShapeDtypeStruct((_N2, _H), f32),
                   jax.ShapeDtypeStruct((2, _H), f32)],
    )(r1, st2, g0.reshape(1, 256), bt0.reshape(1, 256), A1,
      a1b.reshape(1, _H))

    r3, st4 = pl.pallas_call(
        _hk4_body, grid=grid,
        in_specs=[row(_H), stats_spec(_H), full(1, _H), full(1, _H),
                  full(_H, 256), full(1, 256)],
        out_specs=[row(256), stats_spec(256)],
        out_shape=[jax.ShapeDtypeStruct((_N2, 256), f32),
                   jax.ShapeDtypeStruct((2, 256), f32)],
    )(r2, st3, g1.reshape(1, _H), bt1.reshape(1, _H), A2,
      a2b.reshape(1, 256))

    r4, st5 = pl.pallas_call(
        _hk5_body, grid=grid,
        in_specs=[row(256), stats_spec(256), full(1, 256), full(1, 256),
                  row(256), full(256, _W), full(1, _W)],
        out_specs=[row(_W), stats_spec(_W)],
        out_shape=[jax.ShapeDtypeStruct((_N2, _W), f32),
                   jax.ShapeDtypeStruct((2, _W), f32)],
    )(r3, st4, g2.reshape(1, 256), bt2.reshape(1, 256), res1, wl2_p, bl2_p)

    out = pl.pallas_call(
        _hk6_body, grid=grid,
        in_specs=[row(_W), stats_spec(_W), full(1, _W), full(1, _W),
                  row(_W), full(_W, 1), full(1, 1)],
        out_specs=pl.BlockSpec((rb, 1), lambda i: (i, 0)),
        out_shape=jax.ShapeDtypeStruct((_N2, 1), f32),
    )(r4, st5, g3_p, bt3_p, xinb, wl_p, bL.reshape(1, 1))
    return out


# speedup vs baseline: 10.6364x; 1.1296x over previous
import functools
import jax
import jax.numpy as jnp
from jax import lax
from jax.experimental import pallas as pl
from jax.experimental.pallas import tpu as pltpu
from jax.experimental.pallas import tpu_sc as plsc

_N0, _N1, _N2 = 100000, 50000, 10000
_E0, _E1 = 500000, 160000
_D, _H, _G = 128, 128, 1
_DIN = _D + _G + 1
_W = 144
_WF = 64
_WS = 16
_ROWS = 10016
_B = 128
_NB0 = 123
_C0 = _NB0 * _B
_NB1 = 40
_C1 = _NB1 * _B
_STRIPE = _ROWS // 16
_NBUF = 2
_CH = _STRIPE // 2


def _sc_agg_body(nbatch, width, compact, mode, soff,
                 table, src_h, dst_h, zero_h, ones_h,
                 out_h, *rest):
    rest = list(rest)
    rsrc = srcv = rest.pop(0)
    dstv = rest.pop(0)
    csrc = rest.pop(0) if (compact and mode == "feat") else None
    rows = rest.pop(0)
    zbuf = rest.pop(0)
    acc = rest.pop(0)
    sem = rest.pop(0)
    c = lax.axis_index("c")
    s = lax.axis_index("s")
    w = c * 16 + s

    if mode == "feat":
        pltpu.sync_copy(src_h.at[w], srcv)
    else:
        pltpu.sync_copy(ones_h, rows)

    if compact:
        pltpu.sync_copy(dst_h.at[w], dstv.at[pl.ds(0, nbatch)])
    else:
        pltpu.sync_copy(dst_h.at[w], dstv)
    pltpu.sync_copy(zero_h, zbuf)

    def zloop(i, carry):
        pltpu.sync_copy(zbuf, acc.at[pl.ds(s * _STRIPE + i * _CH, _CH)])
        return carry
    lax.fori_loop(0, 2, zloop, 0)

    if compact:
        def cloop(j, cntv):
            drow = dstv.at[j]
            if mode == "feat":
                srow = rsrc.at[j]
            for k in range(_B // 16):
                d16 = drow[pl.ds(k * 16, 16)]
                m = d16 < _N2
                incl = plsc.cumsum(m.astype(jnp.int32))
                pos = jnp.where(m, cntv + incl - 1, nbatch * _B + _B - 1)
                if mode == "feat":
                    s16 = srow[pl.ds(k * 16, 16)]
                    if soff is not None:
                        s16 = (s16 << 1) + soff
                    plsc.store_scatter(csrc, [pos], s16)
                plsc.store_scatter(dstv, [pos >> 7, pos & (_B - 1)], d16)
                cntv = cntv + plsc.all_reduce_population_count(m)
            return cntv
        cntv = lax.fori_loop(0, nbatch, cloop, jnp.zeros((16,), jnp.int32))
        cnt = cntv[0]

        def ploop(i, cend):
            del i
            if mode == "feat":
                csrc[pl.ds(cend, 16)] = jnp.zeros((16,), jnp.int32)
            pp = cend + lax.iota(jnp.int32, 16)
            plsc.store_scatter(dstv, [pp >> 7, pp & (_B - 1)],
                               jnp.full((16,), _N2, jnp.int32))
            return cend + 16
        lax.fori_loop(0, _B // 16, ploop, cnt)

        nb = (cnt + (_B - 1)) // _B
    else:
        nb = nbatch

    plsc.subcore_barrier()

    if mode == "feat":
        def gidx(j):
            if compact:
                return csrc.at[pl.ds(j * _B, _B)]
            return srcv.at[j]

        def fire(j):
            pltpu.async_copy(table.at[gidx(j)], rows.at[j & (_NBUF - 1)], sem)

        def prol(j, carry):
            @pl.when(j < nb)
            def _():
                fire(j)
            return carry
        lax.fori_loop(0, _NBUF, prol, 0)

        def body(j, carry):
            b = j & (_NBUF - 1)
            pltpu.make_async_copy(table.at[gidx(j)], rows.at[b], sem).wait()
            pltpu.sync_copy(rows.at[b], acc.at[dstv.at[j]], add=True)

            @pl.when(j + _NBUF < nb)
            def _():
                fire(j + _NBUF)
            return carry
        lax.fori_loop(0, nb, body, 0)
    else:
        def body(j, carry):
            pltpu.sync_copy(rows, acc.at[dstv.at[j]], add=True)
            return carry
        lax.fori_loop(0, nb, body, 0)

    plsc.subcore_barrier()

    def dump(i, carry):
        r = s * _STRIPE + i * _CH
        pltpu.sync_copy(acc.at[pl.ds(r, _CH)], zbuf)
        pltpu.sync_copy(zbuf, out_h.at[c, pl.ds(r, _CH)])
        return carry
    lax.fori_loop(0, 2, dump, 0)


def _make_sc_agg(nbatch, width, compact, mode, soff=None):
    mesh = plsc.VectorSubcoreMesh(core_axis_name="c", subcore_axis_name="s")
    f32 = jnp.float32
    ne = nbatch * _B
    out_type = [jax.ShapeDtypeStruct((2, _ROWS, width), f32)]
    scratch = [
        pltpu.VMEM((nbatch, _B), jnp.int32),
        pltpu.VMEM((nbatch + (1 if compact else 0), _B), jnp.int32),
    ]
    if compact and mode == "feat":
        scratch.append(pltpu.VMEM((ne + _B,), jnp.int32))
    scratch.append(pltpu.VMEM((_NBUF, _B, width) if mode == "feat"
                              else (_B, width), f32))
    scratch.append(pltpu.VMEM((_CH, width), f32))
    scratch.append(pltpu.VMEM_SHARED((_ROWS, width), f32))
    scratch.append(pltpu.SemaphoreType.DMA)
    return pl.kernel(
        functools.partial(_sc_agg_body, nbatch, width, compact, mode, soff),
        out_type=out_type,
        mesh=mesh,
        scratch_types=scratch,
        compiler_params=pltpu.CompilerParams(
            use_tc_tiling_on_sc=False,
            needs_layout_passes=not compact,
        ),
    )


def _tc1_body(acca_ref, accb_ref, cnt_ref, x_ref, w0n_ref, w0s_ref, b0_ref,
              w1n_ref, h1_ref, saug_ref):
    sa = acca_ref[0] + acca_ref[1]
    sb = accb_ref[0] + accb_ref[1]
    cn = cnt_ref[0] + cnt_ref[1]
    ssum = jnp.concatenate([sa, sb], axis=1)
    cnt = cn[:, 0:1]
    agg = ssum / jnp.maximum(cnt, 1.0)
    h1 = agg @ w0n_ref[...] + x_ref[...] @ w0s_ref[...] + b0_ref[...]
    h1 = jnp.maximum(h1, 0.0)
    h1_ref[...] = h1
    sv = h1 @ w1n_ref[...]
    saug_ref[...] = jnp.concatenate(
        [sv, jnp.ones_like(sv), jnp.zeros((sv.shape[0], _WS - 2), jnp.float32)],
        axis=1)


_N2F = float(_N2)


def _accum_stats(xv, stats_ref):
    part = jnp.concatenate(
        [jnp.sum(xv, axis=0, keepdims=True),
         jnp.sum(xv * xv, axis=0, keepdims=True)], axis=0)

    @pl.when(pl.program_id(0) == 0)
    def _():
        stats_ref[...] = jnp.zeros_like(stats_ref)
    stats_ref[...] += part


def _bn_from_stats(xv, stats, gamma, beta):
    m = stats[0:1] / _N2F
    v = jnp.maximum(stats[1:2] / _N2F - m * m, 0.0)
    return (xv - m) / jnp.sqrt(v + 1e-5) * gamma + beta


def _hk1_body(xnode_ref, h1_ref, acc1_ref, w1s_ref, b1_ref, wl0_ref, bl0_ref,
              wa_ref, ba_ref, xin_ref, stats_ref):
    a1sum = acc1_ref[0] + acc1_ref[1]
    agg1 = a1sum[:, 0:1] / jnp.maximum(a1sum[:, 1:2], 1.0)
    h1 = h1_ref[...]
    h2 = agg1 + h1 @ w1s_ref[...] + b1_ref[...]
    xl0 = h1 @ wl0_ref[...] + bl0_ref[...]
    xin = jnp.concatenate(
        [xnode_ref[...], h2, xl0,
         jnp.zeros((h2.shape[0], _W - _DIN), jnp.float32)], axis=1)
    z = xin @ wa_ref[...] + ba_ref[...]
    z = z - jnp.max(z, axis=1, keepdims=True)
    p = jnp.exp(z)
    p = p / jnp.sum(p, axis=1, keepdims=True)
    xin = xin * p + xin
    xin_ref[...] = xin
    _accum_stats(xin, stats_ref)


def _hk2_body(xin_ref, st_ref, ga_ref, be_ref, a0_ref, a0b_ref,
              xinb_ref, r1_ref, stats_ref):
    xinb = _bn_from_stats(xin_ref[...], st_ref[...], ga_ref[...], be_ref[...])
    xinb_ref[...] = xinb
    r1 = jnp.maximum(xinb @ a0_ref[...] + a0b_ref[...], 0.0)
    r1_ref[...] = r1
    _accum_stats(r1, stats_ref)


def _hk3_body(r1_ref, st_ref, g0_ref, bt0_ref, a1_ref, a1b_ref,
              res1_ref, r2_ref, stats_ref):
    res1 = _bn_from_stats(r1_ref[...], st_ref[...], g0_ref[...], bt0_ref[...])
    res1_ref[...] = res1
    r2 = jnp.maximum(res1 @ a1_ref[...] + a1b_ref[...], 0.0)
    r2_ref[...] = r2
    _accum_stats(r2, stats_ref)


def _hk4_body(r2_ref, st_ref, g1_ref, bt1_ref, a2_ref, a2b_ref,
              r3_ref, stats_ref):
    h = _bn_from_stats(r2_ref[...], st_ref[...], g1_ref[...], bt1_ref[...])
    r3 = jnp.maximum(h @ a2_ref[...] + a2b_ref[...], 0.0)
    r3_ref[...] = r3
    _accum_stats(r3, stats_ref)


def _hk5_body(r3_ref, st_ref, g2_ref, bt2_ref, res1_ref, wl2_ref, bl2_ref,
              r4_ref, stats_ref):
    h = _bn_from_stats(r3_ref[...], st_ref[...], g2_ref[...], bt2_ref[...])
    h = h + res1_ref[...]
    r4 = jnp.maximum(h @ wl2_ref[...] + bl2_ref[...], 0.0)
    r4_ref[...] = r4
    _accum_stats(r4, stats_ref)


def _hk6_body(r4_ref, st_ref, g3_ref, bt3_ref, xinb_ref, wl_ref, bl_ref,
              out_ref):
    h = _bn_from_stats(r4_ref[...], st_ref[...], g3_ref[...], bt3_ref[...])
    h = h + xinb_ref[...]
    out_ref[...] = h @ wl_ref[...] + bl_ref[...]


def _pad_edges(src, dst, per_tile, pad_dst):
    total = 32 * per_tile
    e = src.shape[0]
    src_p = jnp.pad(src, (0, total - e)).reshape(32, per_tile // _B, _B)
    dst_p = jnp.pad(dst, (0, total - e),
                    constant_values=pad_dst).reshape(32, per_tile // _B, _B)
    return src_p, dst_p


@jax.jit
def kernel(x, edge_index_0, edge_index_1, xnode, W0n, W0s, b0, W1n, W1s, b1,
           Wl0, bl0, Wa, ba, ga, be, A0, a0b, g0, bt0, A1, a1b, g1, bt1,
           A2, a2b, g2, bt2, WL2, bL2, g3, bt3, WL, bL):
    f32 = jnp.float32

    xr = x.reshape(2 * _N0, _WF)
    src0, dst0 = _pad_edges(edge_index_0[0], edge_index_0[1], _C0, _N1)
    src1, dst1 = _pad_edges(edge_index_1[0], edge_index_1[1], _C1, _N2)
    zero64 = jnp.zeros((_CH, _WF), f32)
    zero16 = jnp.zeros((_CH, _WS), f32)
    ones16 = jnp.ones((_B, _WS), f32)

    wa_p = jnp.zeros((_W, _W), f32).at[:_DIN, :_DIN].set(Wa)
    ba_p = jnp.full((1, _W), -1e30, f32).at[0, :_DIN].set(ba)
    ga_p = jnp.ones((1, _W), f32).at[0, :_DIN].set(ga)
    be_p = jnp.zeros((1, _W), f32).at[0, :_DIN].set(be)
    a0_p = jnp.zeros((_W, 256), f32).at[:_DIN].set(A0)
    wl2_p = jnp.zeros((256, _W), f32).at[:, :_DIN].set(WL2)
    bl2_p = jnp.zeros((1, _W), f32).at[0, :_DIN].set(bL2)
    g3_p = jnp.ones((1, _W), f32).at[0, :_DIN].set(g3)
    bt3_p = jnp.zeros((1, _W), f32).at[0, :_DIN].set(bt3)
    wl_p = jnp.zeros((_W, 1), f32).at[:_DIN].set(WL)

    acc_a, = _make_sc_agg(_NB0, _WF, True, "feat", 0)(
        xr, src0, dst0, zero64, ones16)
    acc_b, = _make_sc_agg(_NB0, _WF, True, "feat", 1)(
        xr, src0, dst0, zero64, ones16)
    cnt0, = _make_sc_agg(_NB0, _WS, True, "cnt")(
        ones16, src0, dst0, zero16, ones16)

    rb = 1000
    h1, saug = pl.pallas_call(
        _tc1_body,
        grid=(_N2 // rb,),
        in_specs=[
            pl.BlockSpec((2, rb, _WF), lambda i: (0, i, 0)),
            pl.BlockSpec((2, rb, _WF), lambda i: (0, i, 0)),
            pl.BlockSpec((2, rb, _WS), lambda i: (0, i, 0)),
            pl.BlockSpec((rb, _D), lambda i: (i, 0)),
            pl.BlockSpec((_D, _H), lambda i: (0, 0)),
            pl.BlockSpec((_D, _H), lambda i: (0, 0)),
            pl.BlockSpec((1, _H), lambda i: (0, 0)),
            pl.BlockSpec((_H, 1), lambda i: (0, 0)),
        ],
        out_specs=[
            pl.BlockSpec((rb, _H), lambda i: (i, 0)),
            pl.BlockSpec((rb, _WS), lambda i: (i, 0)),
        ],
        out_shape=[
            jax.ShapeDtypeStruct((_N2, _H), f32),
            jax.ShapeDtypeStruct((_N2, _WS), f32),
        ],
    )(acc_a, acc_b, cnt0, x[:_N2], W0n, W0s, b0.reshape(1, _H), W1n)

    acc1, = _make_sc_agg(_NB1, _WS, False, "feat")(
        saug, src1, dst1, zero16, ones16)

    grid = (_N2 // rb,)

    def row(c):
        return pl.BlockSpec((rb, c), lambda i: (i, 0))

    def full(*shape):
        return pl.BlockSpec(shape, lambda i: tuple(0 for _ in shape))

    def stats_spec(c):
        return pl.BlockSpec((2, c), lambda i: (0, 0))

    xin, st1 = pl.pallas_call(
        _hk1_body, grid=grid,
        in_specs=[row(_D), row(_H),
                  pl.BlockSpec((2, rb, _WS), lambda i: (0, i, 0)),
                  full(_H, 1), full(1, 1),
                  full(_H, 1), full(1, 1), full(_W, _W), full(1, _W)],
        out_specs=[row(_W), stats_spec(_W)],
        out_shape=[jax.ShapeDtypeStruct((_N2, _W), f32),
                   jax.ShapeDtypeStruct((2, _W), f32)],
    )(xnode, h1, acc1, W1s, b1.reshape(1, 1), Wl0, bl0.reshape(1, 1),
      wa_p, ba_p)

    xinb, r1, st2 = pl.pallas_call(
        _hk2_body, grid=grid,
        in_specs=[row(_W), stats_spec(_W), full(1, _W), full(1, _W),
                  full(_W, 256), full(1, 256)],
        out_specs=[row(_W), row(256), stats_spec(256)],
        out_shape=[jax.ShapeDtypeStruct((_N2, _W), f32),
                   jax.ShapeDtypeStruct((_N2, 256), f32),
                   jax.ShapeDtypeStruct((2, 256), f32)],
    )(xin, st1, ga_p, be_p, a0_p, a0b.reshape(1, 256))

    res1, r2, st3 = pl.pallas_call(
        _hk3_body, grid=grid,
        in_specs=[row(256), stats_spec(256), full(1, 256), full(1, 256),
                  full(256, _H), full(1, _H)],
        out_specs=[row(256), row(_H), stats_spec(_H)],
        out_shape=[jax.ShapeDtypeStruct((_N2, 256), f32),
                   jax.ShapeDtypeStruct((_N2, _H), f32),
                   jax.ShapeDtypeStruct((2, _H), f32)],
    )(r1, st2, g0.reshape(1, 256), bt0.reshape(1, 256), A1,
      a1b.reshape(1, _H))

    r3, st4 = pl.pallas_call(
        _hk4_body, grid=grid,
        in_specs=[row(_H), stats_spec(_H), full(1, _H), full(1, _H),
                  full(_H, 256), full(1, 256)],
        out_specs=[row(256), stats_spec(256)],
        out_shape=[jax.ShapeDtypeStruct((_N2, 256), f32),
                   jax.ShapeDtypeStruct((2, 256), f32)],
    )(r2, st3, g1.reshape(1, _H), bt1.reshape(1, _H), A2,
      a2b.reshape(1, 256))

    r4, st5 = pl.pallas_call(
        _hk5_body, grid=grid,
        in_specs=[row(256), stats_spec(256), full(1, 256), full(1, 256),
                  row(256), full(256, _W), full(1, _W)],
        out_specs=[row(_W), stats_spec(_W)],
        out_shape=[jax.ShapeDtypeStruct((_N2, _W), f32),
                   jax.ShapeDtypeStruct((2, _W), f32)],
    )(r3, st4, g2.reshape(1, 256), bt2.reshape(1, 256), res1, wl2_p, bl2_p)

    out = pl.pallas_call(
        _hk6_body, grid=grid,
        in_specs=[row(_W), stats_spec(_W), full(1, _W), full(1, _W),
                  row(_W), full(_W, 1), full(1, 1)],
        out_specs=pl.BlockSpec((rb, 1), lambda i: (i, 0)),
        out_shape=jax.ShapeDtypeStruct((_N2, 1), f32),
    )(r4, st5, g3_p, bt3_p, xinb, wl_p, bL.reshape(1, 1))
    return out


# speedup vs baseline: 10.7231x; 1.0081x over previous
import functools
import jax
import jax.numpy as jnp
from jax import lax
from jax.experimental import pallas as pl
from jax.experimental.pallas import tpu as pltpu
from jax.experimental.pallas import tpu_sc as plsc

_N0, _N1, _N2 = 100000, 50000, 10000
_E0, _E1 = 500000, 160000
_D, _H, _G = 128, 128, 1
_DIN = _D + _G + 1
_W = 144
_WF = 64
_WS = 16
_ROWS = 10016
_B = 128
_NB0 = 123
_C0 = _NB0 * _B
_NB1 = 40
_C1 = _NB1 * _B
_STRIPE = _ROWS // 16
_NBUF = 2
_CH = _STRIPE // 2


def _sc_agg_body(nbatch, width, compact, mode, soff,
                 table, src_h, dst_h, zero_h, ones_h,
                 out_h, *rest):
    rest = list(rest)
    rsrc = srcv = rest.pop(0)
    dstv = rest.pop(0)
    csrc = rest.pop(0) if (compact and mode == "feat") else None
    rows = rest.pop(0)
    zbuf = rest.pop(0)
    acc = rest.pop(0)
    sem = rest.pop(0)
    c = lax.axis_index("c")
    s = lax.axis_index("s")
    w = c * 16 + s

    if mode == "feat":
        pltpu.sync_copy(src_h.at[w], srcv)
    else:
        pltpu.sync_copy(ones_h, rows)

    if compact:
        pltpu.sync_copy(dst_h.at[w], dstv.at[pl.ds(0, nbatch)])
    else:
        pltpu.sync_copy(dst_h.at[w], dstv)
    pltpu.sync_copy(zero_h, zbuf)

    def zloop(i, carry):
        pltpu.sync_copy(zbuf, acc.at[pl.ds(s * _STRIPE + i * _CH, _CH)])
        return carry
    lax.fori_loop(0, 2, zloop, 0)

    if compact:
        def cloop(j, cntv):
            drow = dstv.at[j]
            if mode == "feat":
                srow = rsrc.at[j]
            for k in range(_B // 16):
                d16 = drow[pl.ds(k * 16, 16)]
                m = d16 < _N2
                incl = plsc.cumsum(m.astype(jnp.int32))
                pos = jnp.where(m, cntv + incl - 1, nbatch * _B + _B - 1)
                if mode == "feat":
                    s16 = srow[pl.ds(k * 16, 16)]
                    if soff is not None:
                        s16 = (s16 << 1) + soff
                    plsc.store_scatter(csrc, [pos], s16)
                plsc.store_scatter(dstv, [pos >> 7, pos & (_B - 1)], d16)
                cntv = cntv + plsc.all_reduce_population_count(m)
            return cntv
        cntv = lax.fori_loop(0, nbatch, cloop, jnp.zeros((16,), jnp.int32))
        cnt = cntv[0]

        def ploop(i, cend):
            del i
            if mode == "feat":
                csrc[pl.ds(cend, 16)] = jnp.zeros((16,), jnp.int32)
            pp = cend + lax.iota(jnp.int32, 16)
            plsc.store_scatter(dstv, [pp >> 7, pp & (_B - 1)],
                               jnp.full((16,), _N2, jnp.int32))
            return cend + 16
        lax.fori_loop(0, _B // 16, ploop, cnt)

        nb = (cnt + (_B - 1)) // _B
    else:
        nb = nbatch

    plsc.subcore_barrier()

    if mode == "feat":
        def gidx(j):
            if compact:
                return csrc.at[pl.ds(j * _B, _B)]
            return srcv.at[j]

        def fire(j):
            pltpu.async_copy(table.at[gidx(j)], rows.at[j & (_NBUF - 1)], sem)

        def prol(j, carry):
            @pl.when(j < nb)
            def _():
                fire(j)
            return carry
        lax.fori_loop(0, _NBUF, prol, 0)

        def body(j, carry):
            b = j & (_NBUF - 1)
            pltpu.make_async_copy(table.at[gidx(j)], rows.at[b], sem).wait()
            pltpu.sync_copy(rows.at[b], acc.at[dstv.at[j]], add=True)

            @pl.when(j + _NBUF < nb)
            def _():
                fire(j + _NBUF)
            return carry
        lax.fori_loop(0, nb, body, 0)
    else:
        def body(j, carry):
            pltpu.sync_copy(rows, acc.at[dstv.at[j]], add=True)
            return carry
        lax.fori_loop(0, nb, body, 0)

    plsc.subcore_barrier()

    def dump(i, carry):
        r = s * _STRIPE + i * _CH
        pltpu.sync_copy(acc.at[pl.ds(r, _CH)], zbuf)
        pltpu.sync_copy(zbuf, out_h.at[c, pl.ds(r, _CH)])
        return carry
    lax.fori_loop(0, 2, dump, 0)


def _make_sc_agg(nbatch, width, compact, mode, soff=None):
    mesh = plsc.VectorSubcoreMesh(core_axis_name="c", subcore_axis_name="s")
    f32 = jnp.float32
    ne = nbatch * _B
    out_type = [jax.ShapeDtypeStruct((2, _ROWS, width), f32)]
    scratch = [
        pltpu.VMEM((nbatch, _B), jnp.int32),
        pltpu.VMEM((nbatch + (1 if compact else 0), _B), jnp.int32),
    ]
    if compact and mode == "feat":
        scratch.append(pltpu.VMEM((ne + _B,), jnp.int32))
    scratch.append(pltpu.VMEM((_NBUF, _B, width) if mode == "feat"
                              else (_B, width), f32))
    scratch.append(pltpu.VMEM((_CH, width), f32))
    scratch.append(pltpu.VMEM_SHARED((_ROWS, width), f32))
    scratch.append(pltpu.SemaphoreType.DMA)
    return pl.kernel(
        functools.partial(_sc_agg_body, nbatch, width, compact, mode, soff),
        out_type=out_type,
        mesh=mesh,
        scratch_types=scratch,
        compiler_params=pltpu.CompilerParams(
            use_tc_tiling_on_sc=False,
            needs_layout_passes=not compact,
        ),
    )


def _tc1_body(acca_ref, accb_ref, cnt_ref, x_ref, w0n_ref, w0s_ref, b0_ref,
              w1n_ref, h1_ref, saug_ref):
    sa = acca_ref[0] + acca_ref[1]
    sb = accb_ref[0] + accb_ref[1]
    cn = cnt_ref[0] + cnt_ref[1]
    ssum = jnp.concatenate([sa, sb], axis=1)
    cnt = cn[:, 0:1]
    agg = ssum / jnp.maximum(cnt, 1.0)
    h1 = agg @ w0n_ref[...] + x_ref[...] @ w0s_ref[...] + b0_ref[...]
    h1 = jnp.maximum(h1, 0.0)
    h1_ref[...] = h1
    sv = h1 @ w1n_ref[...]
    saug_ref[...] = jnp.concatenate(
        [sv, jnp.ones_like(sv), jnp.zeros((sv.shape[0], _WS - 2), jnp.float32)],
        axis=1)


_N2F = float(_N2)


def _accum_stats(xv, stats_ref):
    part = jnp.concatenate(
        [jnp.sum(xv, axis=0, keepdims=True),
         jnp.sum(xv * xv, axis=0, keepdims=True)], axis=0)

    @pl.when(pl.program_id(0) == 0)
    def _():
        stats_ref[...] = jnp.zeros_like(stats_ref)
    stats_ref[...] += part


def _bn_from_stats(xv, stats, gamma, beta):
    m = stats[0:1] / _N2F
    v = jnp.maximum(stats[1:2] / _N2F - m * m, 0.0)
    return (xv - m) / jnp.sqrt(v + 1e-5) * gamma + beta


def _hk1_body(xnode_ref, h1_ref, acc1_ref, w1s_ref, b1_ref, wl0_ref, bl0_ref,
              wa_ref, ba_ref, xin_ref, stats_ref):
    a1sum = acc1_ref[0] + acc1_ref[1]
    agg1 = a1sum[:, 0:1] / jnp.maximum(a1sum[:, 1:2], 1.0)
    h1 = h1_ref[...]
    h2 = agg1 + h1 @ w1s_ref[...] + b1_ref[...]
    xl0 = h1 @ wl0_ref[...] + bl0_ref[...]
    xin = jnp.concatenate(
        [xnode_ref[...], h2, xl0,
         jnp.zeros((h2.shape[0], _W - _DIN), jnp.float32)], axis=1)
    z = xin @ wa_ref[...] + ba_ref[...]
    z = z - jnp.max(z, axis=1, keepdims=True)
    p = jnp.exp(z)
    p = p / jnp.sum(p, axis=1, keepdims=True)
    xin = xin * p + xin
    xin_ref[...] = xin
    _accum_stats(xin, stats_ref)


def _mega_body(xin_ref, st1_ref, w_ref, b_ref, g_ref, bt_ref,
               out_ref, cur, xinb_s, res1_s, statsS):
    p = pl.program_id(0)
    i = pl.program_id(1)
    rb = xin_ref.shape[0]
    blk = pl.ds(i * rb, rb)

    @pl.when((p == 0) & (i == 0))
    def _():
        statsS[0] = st1_ref[...]

    xpad = jnp.concatenate(
        [xin_ref[...], jnp.zeros((rb, 256 - _W), jnp.float32)], axis=1)
    xv = jnp.where(p == 0, xpad, cur[blk, :])
    st = statsS[p]
    m = st[0:1] / _N2F
    v = jnp.maximum(st[1:2] / _N2F - m * m, 0.0)
    xb = (xv - m) / jnp.sqrt(v + 1e-5) * g_ref[0] + bt_ref[0]

    @pl.when(p == 0)
    def _():
        xinb_s[blk, :] = xb

    @pl.when(p == 1)
    def _():
        res1_s[blk, :] = xb

    addv = jnp.where(p == 3, res1_s[blk, :],
                     jnp.where(p == 4, xinb_s[blk, :],
                               jnp.zeros((rb, 256), jnp.float32)))
    hb = xb + addv
    y = hb @ w_ref[0] + b_ref[0]
    z = jnp.where(p < 4, jnp.maximum(y, 0.0), y)
    cur[blk, :] = z
    part = jnp.concatenate(
        [jnp.sum(z, axis=0, keepdims=True),
         jnp.sum(z * z, axis=0, keepdims=True)], axis=0)

    @pl.when(i == 0)
    def _():
        statsS[p + 1] = part

    @pl.when(i != 0)
    def _():
        statsS[p + 1] += part

    out_ref[...] = y[:, 0:1]


def _pad_edges(src, dst, per_tile, pad_dst):
    total = 32 * per_tile
    e = src.shape[0]
    src_p = jnp.pad(src, (0, total - e)).reshape(32, per_tile // _B, _B)
    dst_p = jnp.pad(dst, (0, total - e),
                    constant_values=pad_dst).reshape(32, per_tile // _B, _B)
    return src_p, dst_p


@jax.jit
def kernel(x, edge_index_0, edge_index_1, xnode, W0n, W0s, b0, W1n, W1s, b1,
           Wl0, bl0, Wa, ba, ga, be, A0, a0b, g0, bt0, A1, a1b, g1, bt1,
           A2, a2b, g2, bt2, WL2, bL2, g3, bt3, WL, bL):
    f32 = jnp.float32

    xr = x.reshape(2 * _N0, _WF)
    src0, dst0 = _pad_edges(edge_index_0[0], edge_index_0[1], _C0, _N1)
    src1, dst1 = _pad_edges(edge_index_1[0], edge_index_1[1], _C1, _N2)
    zero64 = jnp.zeros((_CH, _WF), f32)
    zero16 = jnp.zeros((_CH, _WS), f32)
    ones16 = jnp.ones((_B, _WS), f32)

    wa_p = jnp.zeros((_W, _W), f32).at[:_DIN, :_DIN].set(Wa)
    ba_p = jnp.full((1, _W), -1e30, f32).at[0, :_DIN].set(ba)
    ga_p = jnp.ones((1, _W), f32).at[0, :_DIN].set(ga)
    be_p = jnp.zeros((1, _W), f32).at[0, :_DIN].set(be)
    a0_p = jnp.zeros((_W, 256), f32).at[:_DIN].set(A0)
    wl2_p = jnp.zeros((256, _W), f32).at[:, :_DIN].set(WL2)
    bl2_p = jnp.zeros((1, _W), f32).at[0, :_DIN].set(bL2)
    g3_p = jnp.ones((1, _W), f32).at[0, :_DIN].set(g3)
    bt3_p = jnp.zeros((1, _W), f32).at[0, :_DIN].set(bt3)
    wl_p = jnp.zeros((_W, 1), f32).at[:_DIN].set(WL)

    acc_a, = _make_sc_agg(_NB0, _WF, True, "feat", 0)(
        xr, src0, dst0, zero64, ones16)
    acc_b, = _make_sc_agg(_NB0, _WF, True, "feat", 1)(
        xr, src0, dst0, zero64, ones16)
    cnt0, = _make_sc_agg(_NB0, _WS, True, "cnt")(
        ones16, src0, dst0, zero16, ones16)

    rb = 1000
    h1, saug = pl.pallas_call(
        _tc1_body,
        grid=(_N2 // rb,),
        in_specs=[
            pl.BlockSpec((2, rb, _WF), lambda i: (0, i, 0)),
            pl.BlockSpec((2, rb, _WF), lambda i: (0, i, 0)),
            pl.BlockSpec((2, rb, _WS), lambda i: (0, i, 0)),
            pl.BlockSpec((rb, _D), lambda i: (i, 0)),
            pl.BlockSpec((_D, _H), lambda i: (0, 0)),
            pl.BlockSpec((_D, _H), lambda i: (0, 0)),
            pl.BlockSpec((1, _H), lambda i: (0, 0)),
            pl.BlockSpec((_H, 1), lambda i: (0, 0)),
        ],
        out_specs=[
            pl.BlockSpec((rb, _H), lambda i: (i, 0)),
            pl.BlockSpec((rb, _WS), lambda i: (i, 0)),
        ],
        out_shape=[
            jax.ShapeDtypeStruct((_N2, _H), f32),
            jax.ShapeDtypeStruct((_N2, _WS), f32),
        ],
    )(acc_a, acc_b, cnt0, x[:_N2], W0n, W0s, b0.reshape(1, _H), W1n)

    acc1, = _make_sc_agg(_NB1, _WS, False, "feat")(
        saug, src1, dst1, zero16, ones16)

    grid = (_N2 // rb,)

    def row(c):
        return pl.BlockSpec((rb, c), lambda i: (i, 0))

    def full(*shape):
        return pl.BlockSpec(shape, lambda i: tuple(0 for _ in shape))

    def stats_spec(c):
        return pl.BlockSpec((2, c), lambda i: (0, 0))

    xin, st1 = pl.pallas_call(
        _hk1_body, grid=grid,
        in_specs=[row(_D), row(_H),
                  pl.BlockSpec((2, rb, _WS), lambda i: (0, i, 0)),
                  full(_H, 1), full(1, 1),
                  full(_H, 1), full(1, 1), full(_W, _W), full(1, _W)],
        out_specs=[row(_W), stats_spec(_W)],
        out_shape=[jax.ShapeDtypeStruct((_N2, _W), f32),
                   jax.ShapeDtypeStruct((2, _W), f32)],
    )(xnode, h1, acc1, W1s, b1.reshape(1, 1), Wl0, bl0.reshape(1, 1),
      wa_p, ba_p)

    st1_p = jnp.zeros((2, 256), f32).at[:, :_W].set(st1)
    wstack = jnp.zeros((5, 256, 256), f32)
    wstack = wstack.at[0, :_W, :].set(a0_p)
    wstack = wstack.at[1, :, :128].set(A1)
    wstack = wstack.at[2, :128, :].set(A2)
    wstack = wstack.at[3, :, :_W].set(wl2_p)
    wstack = wstack.at[4, :_W, 0:1].set(wl_p)
    bstack = jnp.zeros((5, 1, 256), f32)
    bstack = bstack.at[0, 0, :256].set(a0b)
    bstack = bstack.at[1, 0, :128].set(a1b)
    bstack = bstack.at[2, 0, :256].set(a2b)
    bstack = bstack.at[3, 0, :_W].set(bl2_p[0])
    bstack = bstack.at[4, 0, 0].set(bL[0])
    gstack = jnp.ones((5, 1, 256), f32)
    gstack = gstack.at[0, 0, :_W].set(ga_p[0])
    gstack = gstack.at[1, 0, :256].set(g0)
    gstack = gstack.at[2, 0, :128].set(g1)
    gstack = gstack.at[3, 0, :256].set(g2)
    gstack = gstack.at[4, 0, :_W].set(g3_p[0])
    btstack = jnp.zeros((5, 1, 256), f32)
    btstack = btstack.at[0, 0, :_W].set(be_p[0])
    btstack = btstack.at[1, 0, :256].set(bt0)
    btstack = btstack.at[2, 0, :128].set(bt1)
    btstack = btstack.at[3, 0, :256].set(bt2)
    btstack = btstack.at[4, 0, :_W].set(bt3_p[0])

    out = pl.pallas_call(
        _mega_body,
        grid=(5, _N2 // rb),
        in_specs=[
            pl.BlockSpec((rb, _W), lambda p, i: (i, 0)),
            pl.BlockSpec((2, 256), lambda p, i: (0, 0)),
            pl.BlockSpec((1, 256, 256), lambda p, i: (p, 0, 0)),
            pl.BlockSpec((1, 1, 256), lambda p, i: (p, 0, 0)),
            pl.BlockSpec((1, 1, 256), lambda p, i: (p, 0, 0)),
            pl.BlockSpec((1, 1, 256), lambda p, i: (p, 0, 0)),
        ],
        out_specs=pl.BlockSpec((rb, 1), lambda p, i: (i, 0)),
        out_shape=jax.ShapeDtypeStruct((_N2, 1), f32),
        scratch_shapes=[
            pltpu.VMEM((_N2, 256), f32),
            pltpu.VMEM((_N2, 256), f32),
            pltpu.VMEM((_N2, 256), f32),
            pltpu.VMEM((6, 2, 256), f32),
        ],
    )(xin, st1_p, wstack, bstack, gstack, btstack)
    return out
